# baseline (speedup 1.0000x reference)
import functools

import jax
import jax.numpy as jnp
from jax import lax
from jax.experimental import pallas as pl
from jax.experimental.pallas import tpu as pltpu
from jax.experimental.pallas import tpu_sc as plsc

D_MODEL = 768
EPS = 1e-5
NCORES = 2
NSUBCORES = 16
NWORKERS = NCORES * NSUBCORES
CHUNK = 64
TC_BLOCK = 2048
NSLICES = 4


def _make_sc_gather(total_tokens):
    tok_per_w = total_tokens // NWORKERS
    nchunks = tok_per_w // CHUNK
    mesh = plsc.VectorSubcoreMesh(core_axis_name="c", subcore_axis_name="s")

    @functools.partial(
        pl.kernel,
        out_type=jax.ShapeDtypeStruct((total_tokens, D_MODEL), jnp.float32),
        mesh=mesh,
        scratch_types=[
            pltpu.VMEM((tok_per_w,), jnp.int32),
            pltpu.VMEM((2, CHUNK, D_MODEL), jnp.float32),
            pltpu.SemaphoreType.DMA((2,)),
            pltpu.SemaphoreType.DMA((2,)),
        ],
    )
    def k(ids_hbm, word_hbm, out_hbm, idx_v, rows_v, sem_g, sem_o):
        wid = lax.axis_index("s") * NCORES + lax.axis_index("c")
        tstart = wid * tok_per_w
        pltpu.sync_copy(ids_hbm.at[pl.ds(tstart, tok_per_w)], idx_v)

        def gather_copy(c):
            return pltpu.make_async_copy(
                word_hbm.at[idx_v.at[pl.ds(c * CHUNK, CHUNK)]],
                rows_v.at[lax.rem(c, 2)], sem_g.at[lax.rem(c, 2)])

        def out_copy(c):
            return pltpu.make_async_copy(
                rows_v.at[lax.rem(c, 2)],
                out_hbm.at[pl.ds(tstart + c * CHUNK, CHUNK), :],
                sem_o.at[lax.rem(c, 2)])

        gather_copy(0).start()

        def chunk_body(c, _):
            @pl.when(c >= 1)
            def _():
                out_copy(c - 1).wait()

            @pl.when(c + 1 < nchunks)
            def _():
                gather_copy(c + 1).start()

            gather_copy(c).wait()
            out_copy(c).start()
            return 0

        lax.fori_loop(0, nchunks, chunk_body, 0)
        out_copy(nchunks - 1).wait()

    return k


def _tc_ln_body_first(rows_ref, p_ref, gam_ref, bet_ref, o_ref):
    x = rows_ref[0] + p_ref[...]
    mu = jnp.mean(x, axis=-1, keepdims=True)
    var = jnp.mean(x * x, axis=-1, keepdims=True) - mu * mu
    o_ref[0] = (x - mu) * lax.rsqrt(var + EPS) * gam_ref[...] + bet_ref[...]


def _tc_ln_body(rows_ref, p_ref, gam_ref, bet_ref, big_ref, o_ref):
    del big_ref
    _tc_ln_body_first(rows_ref, p_ref, gam_ref, bet_ref, o_ref)


def _make_tc_ln(batch, seq_len, s_per_slice, s_offset, first):
    blk = s_per_slice if s_per_slice <= TC_BLOCK else s_per_slice // 2
    nsb = s_per_slice // blk
    sb0 = s_offset // blk

    in_specs = [
        pl.BlockSpec((1, blk, D_MODEL), lambda s, b: (b, s, 0)),
        pl.BlockSpec((blk, D_MODEL), lambda s, b: (sb0 + s, 0)),
        pl.BlockSpec((1, D_MODEL), lambda s, b: (0, 0)),
        pl.BlockSpec((1, D_MODEL), lambda s, b: (0, 0)),
    ]
    if first:
        body = _tc_ln_body_first
        aliases = {}
    else:
        body = _tc_ln_body
        in_specs.append(pl.BlockSpec(memory_space=pl.ANY))
        aliases = {4: 0}
    return pl.pallas_call(
        body,
        grid=(nsb, batch),
        in_specs=in_specs,
        out_specs=pl.BlockSpec((1, blk, D_MODEL),
                               lambda s, b: (b, sb0 + s, 0)),
        out_shape=jax.ShapeDtypeStruct((batch, seq_len, D_MODEL), jnp.float32),
        input_output_aliases=aliases,
    )


def kernel(input_ids, word_table, pos_table, gamma, beta):
    batch, seq_len = input_ids.shape
    ids = input_ids.astype(jnp.int32)
    if seq_len == 8192 and batch == 4:
        slice_sizes = [2048, 2048, 2048, 2048]
    else:
        slice_sizes = [seq_len // NSLICES] * NSLICES
    gathers = {n: _make_sc_gather(batch * n) for n in set(slice_sizes)}
    gam2 = gamma.reshape(1, D_MODEL)
    bet2 = beta.reshape(1, D_MODEL)

    rows = []
    offs = []
    s0 = 0
    for n in slice_sizes:
        ids_k = lax.slice_in_dim(ids, s0, s0 + n,
                                 axis=1).reshape(batch * n)
        rows.append(gathers[n](ids_k, word_table)
                    .reshape(batch, n, D_MODEL))
        offs.append(s0)
        s0 += n

    big = _make_tc_ln(batch, seq_len, slice_sizes[0], 0, True)(
        rows[0], pos_table, gam2, bet2)
    for k in range(1, len(slice_sizes)):
        big = _make_tc_ln(batch, seq_len, slice_sizes[k], offs[k], False)(
            rows[k], pos_table, gam2, bet2, big)
    return big


# speedup vs baseline: 1.0044x; 1.0044x over previous
import functools

import jax
import jax.numpy as jnp
from jax import lax
from jax.experimental import pallas as pl
from jax.experimental.pallas import tpu as pltpu
from jax.experimental.pallas import tpu_sc as plsc

D_MODEL = 768
EPS = 1e-5
NCORES = 2
NSUBCORES = 16
NWORKERS = NCORES * NSUBCORES
CHUNK = 64
TC_BLOCK = 2048
NSLICES = 4


def _make_sc_gather(total_tokens):
    tok_per_w = total_tokens // NWORKERS
    nchunks = tok_per_w // CHUNK
    mesh = plsc.VectorSubcoreMesh(core_axis_name="c", subcore_axis_name="s")

    @functools.partial(
        pl.kernel,
        out_type=jax.ShapeDtypeStruct((total_tokens, D_MODEL), jnp.float32),
        mesh=mesh,
        scratch_types=[
            pltpu.VMEM((tok_per_w,), jnp.int32),
            pltpu.VMEM((2, CHUNK, D_MODEL), jnp.float32),
            pltpu.SemaphoreType.DMA((2,)),
            pltpu.SemaphoreType.DMA((2,)),
        ],
    )
    def k(ids_hbm, word_hbm, out_hbm, idx_v, rows_v, sem_g, sem_o):
        wid = lax.axis_index("s") * NCORES + lax.axis_index("c")
        tstart = wid * tok_per_w
        pltpu.sync_copy(ids_hbm.at[pl.ds(tstart, tok_per_w)], idx_v)

        def gather_copy(c):
            return pltpu.make_async_copy(
                word_hbm.at[idx_v.at[pl.ds(c * CHUNK, CHUNK)]],
                rows_v.at[lax.rem(c, 2)], sem_g.at[lax.rem(c, 2)])

        def out_copy(c):
            return pltpu.make_async_copy(
                rows_v.at[lax.rem(c, 2)],
                out_hbm.at[pl.ds(tstart + c * CHUNK, CHUNK), :],
                sem_o.at[lax.rem(c, 2)])

        gather_copy(0).start()

        def chunk_body(c, _):
            @pl.when(c >= 1)
            def _():
                out_copy(c - 1).wait()

            @pl.when(c + 1 < nchunks)
            def _():
                gather_copy(c + 1).start()

            gather_copy(c).wait()
            out_copy(c).start()
            return 0

        lax.fori_loop(0, nchunks, chunk_body, 0)
        out_copy(nchunks - 1).wait()

    return k


def _tc_ln_body_first(rows_ref, p_ref, gam_ref, bet_ref, o_ref):
    x = rows_ref[0] + p_ref[...]
    mu = jnp.mean(x, axis=-1, keepdims=True)
    var = jnp.mean(x * x, axis=-1, keepdims=True) - mu * mu
    o_ref[0] = (x - mu) * lax.rsqrt(var + EPS) * gam_ref[...] + bet_ref[...]


def _tc_ln_body(rows_ref, p_ref, gam_ref, bet_ref, big_ref, o_ref):
    del big_ref
    _tc_ln_body_first(rows_ref, p_ref, gam_ref, bet_ref, o_ref)


def _make_tc_ln(batch, seq_len, s_per_slice, s_offset, first):
    blk = s_per_slice if s_per_slice <= TC_BLOCK else s_per_slice // 2
    nsb = s_per_slice // blk
    sb0 = s_offset // blk

    in_specs = [
        pl.BlockSpec((1, blk, D_MODEL), lambda s, b: (b, s, 0)),
        pl.BlockSpec((blk, D_MODEL), lambda s, b: (sb0 + s, 0)),
        pl.BlockSpec((1, D_MODEL), lambda s, b: (0, 0)),
        pl.BlockSpec((1, D_MODEL), lambda s, b: (0, 0)),
    ]
    if first:
        body = _tc_ln_body_first
        aliases = {}
    else:
        body = _tc_ln_body
        in_specs.append(pl.BlockSpec(memory_space=pl.ANY))
        aliases = {4: 0}
    return pl.pallas_call(
        body,
        grid=(nsb, batch),
        in_specs=in_specs,
        out_specs=pl.BlockSpec((1, blk, D_MODEL),
                               lambda s, b: (b, sb0 + s, 0)),
        out_shape=jax.ShapeDtypeStruct((batch, seq_len, D_MODEL), jnp.float32),
        input_output_aliases=aliases,
    )


def kernel(input_ids, word_table, pos_table, gamma, beta):
    batch, seq_len = input_ids.shape
    ids = input_ids.astype(jnp.int32)
    slice_sizes = [seq_len // NSLICES] * NSLICES
    gathers = {n: _make_sc_gather(batch * n) for n in set(slice_sizes)}
    gam2 = gamma.reshape(1, D_MODEL)
    bet2 = beta.reshape(1, D_MODEL)

    rows = []
    offs = []
    s0 = 0
    for n in slice_sizes:
        ids_k = lax.slice_in_dim(ids, s0, s0 + n,
                                 axis=1).reshape(batch * n)
        rows.append(gathers[n](ids_k, word_table)
                    .reshape(batch, n, D_MODEL))
        offs.append(s0)
        s0 += n

    big = _make_tc_ln(batch, seq_len, slice_sizes[0], 0, True)(
        rows[0], pos_table, gam2, bet2)
    for k in range(1, len(slice_sizes)):
        big = _make_tc_ln(batch, seq_len, slice_sizes[k], offs[k], False)(
            rows[k], pos_table, gam2, bet2, big)
    return big
